# baseline (speedup 1.0000x reference)
import jax
import jax.numpy as jnp
from jax.experimental import pallas as pl
from jax.experimental.pallas import tpu as pltpu


_DN_T = (((1,), (1,)), ((), ()))


def _fused_kernel(y_ref, z_ref, wy_ref, wz_ref, b_ref, out_ref,
                  wyb_s, wzb_s):
    @pl.when(pl.program_id(1) == 0)
    def _cast_weights():
        wyb_s[...] = wy_ref[...].astype(jnp.bfloat16)
        wzb_s[...] = wz_ref[...].astype(jnp.bfloat16)

    yb = y_ref[...].astype(jnp.bfloat16)
    zb = z_ref[...].astype(jnp.bfloat16)
    acc = jax.lax.dot_general(yb, wyb_s[...], _DN_T,
                              preferred_element_type=jnp.float32)
    acc = acc + jax.lax.dot_general(zb, wzb_s[...], _DN_T,
                                    preferred_element_type=jnp.float32)
    out_ref[...] = acc + b_ref[...]


def kernel(y, z, weight_y, weight_z, bias, *, tile_n=1024, num_cores=2):
    n, yin = y.shape
    zin = z.shape[1]
    out_dim = weight_y.shape[0]

    b_row = bias.astype(jnp.float32).reshape(1, out_dim)

    n_tiles = n // tile_n
    inner = n_tiles // num_cores
    grid = (num_cores, inner)

    bytes_accessed = (
        y.size * 4 + z.size * 4
        + weight_y.size * 4 + weight_z.size * 4
        + b_row.size * 4
        + n * out_dim * 4
    )

    out = pl.pallas_call(
        _fused_kernel,
        out_shape=jax.ShapeDtypeStruct((n, out_dim), jnp.float32),
        grid=grid,
        in_specs=[
            pl.BlockSpec((tile_n, yin), lambda i, j, k=inner: (i * k + j, 0)),
            pl.BlockSpec((tile_n, zin), lambda i, j, k=inner: (i * k + j, 0)),
            pl.BlockSpec((out_dim, yin), lambda i, j: (0, 0)),
            pl.BlockSpec((out_dim, zin), lambda i, j: (0, 0)),
            pl.BlockSpec((1, out_dim), lambda i, j: (0, 0)),
        ],
        out_specs=pl.BlockSpec((tile_n, out_dim),
                               lambda i, j, k=inner: (i * k + j, 0)),
        scratch_shapes=[
            pltpu.VMEM((out_dim, yin), jnp.bfloat16),
            pltpu.VMEM((out_dim, zin), jnp.bfloat16),
        ],
        compiler_params=pltpu.CompilerParams(
            dimension_semantics=("parallel", "arbitrary"),
        ),
        cost_estimate=pl.CostEstimate(
            flops=2 * n * (yin + zin) * out_dim,
            transcendentals=0,
            bytes_accessed=bytes_accessed,
        ),
    )(y, z, weight_y, weight_z, b_row)
    return out


# speedup vs baseline: 1.0175x; 1.0175x over previous
import jax
import jax.numpy as jnp
from jax.experimental import pallas as pl
from jax.experimental.pallas import tpu as pltpu


_DN_T = (((1,), (1,)), ((), ()))


def _fused_kernel(y_ref, z_ref, wy_ref, wz_ref, b_ref, out_ref):
    yb = y_ref[...].astype(jnp.bfloat16)
    zb = z_ref[...].astype(jnp.bfloat16)
    wyb = wy_ref[...].astype(jnp.bfloat16)
    wzb = wz_ref[...].astype(jnp.bfloat16)
    acc = jax.lax.dot_general(yb, wyb, _DN_T, preferred_element_type=jnp.float32)
    acc = acc + jax.lax.dot_general(zb, wzb, _DN_T, preferred_element_type=jnp.float32)
    out_ref[...] = acc + b_ref[...]


def kernel(y, z, weight_y, weight_z, bias, *, tile_n=1024):
    n, yin = y.shape
    zin = z.shape[1]
    out_dim = weight_y.shape[0]

    b_row = bias.astype(jnp.float32).reshape(1, out_dim)

    grid = (n // tile_n,)

    bytes_accessed = (
        y.size * 4 + z.size * 4
        + weight_y.size * 4 + weight_z.size * 4
        + b_row.size * 4
        + n * out_dim * 4
    )

    out = pl.pallas_call(
        _fused_kernel,
        out_shape=jax.ShapeDtypeStruct((n, out_dim), jnp.float32),
        grid=grid,
        in_specs=[
            pl.BlockSpec((tile_n, yin), lambda i: (i, 0)),
            pl.BlockSpec((tile_n, zin), lambda i: (i, 0)),
            pl.BlockSpec((out_dim, yin), lambda i: (0, 0)),
            pl.BlockSpec((out_dim, zin), lambda i: (0, 0)),
            pl.BlockSpec((1, out_dim), lambda i: (0, 0)),
        ],
        out_specs=pl.BlockSpec((tile_n, out_dim), lambda i: (i, 0)),
        compiler_params=pltpu.CompilerParams(
            dimension_semantics=("parallel",),
        ),
        cost_estimate=pl.CostEstimate(
            flops=2 * n * (yin + zin) * out_dim,
            transcendentals=0,
            bytes_accessed=bytes_accessed,
        ),
    )(y, z, weight_y, weight_z, b_row)
    return out
